# Initial kernel scaffold; baseline (speedup 1.0000x reference)
import jax
import jax.numpy as jnp
from jax.experimental import pallas as pl


def kernel(x, enc_conv1_w, enc_conv1_s, enc_conv1_h, enc_conv2_w, enc_conv2_s, enc_conv2_h, enc_conv3_w, enc_conv3_s, enc_conv3_h, enc_pc2_w, enc_pc2_b, enc_pc3_w, enc_pc3_b, cls_conv1_w, cls_conv1_s, cls_conv1_h, cls_conv2_w, cls_conv2_s, cls_conv2_h, cls_conv3_w, cls_conv3_s, cls_conv3_h, cls_pc2_w, cls_pc2_b, cls_pc3_w, cls_pc3_b, cls_fc_w, cls_fc_b, cls_fc1_w, cls_fc1_b):
    raise NotImplementedError("write your pallas kernel here")


# speedup vs baseline: 1.0900x; 1.0900x over previous
import jax
import jax.numpy as jnp
from jax.experimental import pallas as pl
from jax.experimental.pallas import tpu as pltpu


def _plen(l):
    return (l - 5) // 2 + 1


def _pool5s2(ref, lin):
    p = _plen(lin)
    m = ref[pl.ds(0, p, 2), :]
    for k in range(1, 5):
        m = jnp.maximum(m, ref[pl.ds(k, p, 2), :])
    return m


def _im2col5(pad_ref, lout):
    cols = [pad_ref[k:k + lout, :] for k in range(5)]
    return jnp.concatenate(cols, axis=1).astype(jnp.bfloat16)


def _fuse_conv_point(wc, scale, wp):
    cin, co = wc.shape[1], wc.shape[2]
    conv_part = (wc * scale).reshape(5 * cin, co)
    point_part = jnp.zeros((5 * cin, wp.shape[1]), jnp.float32)
    point_part = point_part.at[2 * cin:3 * cin].set(wp)
    return jnp.concatenate([conv_part, point_part], axis=1).astype(jnp.bfloat16)


def _make_enc_kernel(L):
    P1, P2 = _plen(L), _plen(_plen(L))

    def body(x_ref, w1_ref, h1_ref, w2_ref, h2_ref, bp2_ref,
             w3_ref, h3_ref, bp3_ref, o_ref,
             pad1, act1, pad2, act2, pad3, act3):
        pad1[0:2, :] = jnp.zeros((2, 1), jnp.float32)
        pad1[L + 2:L + 4, :] = jnp.zeros((2, 1), jnp.float32)
        pad1[2:L + 2, :] = x_ref[0]
        acc = pad1[0:L, :] * w1_ref[0:1, :]
        for k in range(1, 5):
            acc = acc + pad1[k:k + L, :] * w1_ref[k:k + 1, :]
        act1[...] = jnp.maximum(acc + h1_ref[...], 0.0)

        c1 = w1_ref.shape[1]
        pad2[0:2, :] = jnp.zeros((2, c1), jnp.float32)
        pad2[P1 + 2:P1 + 4, :] = jnp.zeros((2, c1), jnp.float32)
        pad2[2:P1 + 2, :] = _pool5s2(act1, L)
        big = jnp.dot(_im2col5(pad2, P1), w2_ref[...],
                      preferred_element_type=jnp.float32)
        c2 = big.shape[1] // 2
        act2[...] = (big[:, c2:] + bp2_ref[...]
                     + jnp.maximum(big[:, :c2] + h2_ref[...], 0.0))

        pad3[0:2, :] = jnp.zeros((2, c2), jnp.float32)
        pad3[P2 + 2:P2 + 4, :] = jnp.zeros((2, c2), jnp.float32)
        pad3[2:P2 + 2, :] = _pool5s2(act2, P1)
        big3 = jnp.dot(_im2col5(pad3, P2), w3_ref[...],
                       preferred_element_type=jnp.float32)
        c3 = big3.shape[1] // 2
        act3[...] = (big3[:, c3:] + bp3_ref[...]
                     + jnp.maximum(big3[:, :c3] + h3_ref[...], 0.0))

        o_ref[0] = jnp.mean(_pool5s2(act3, P2), axis=0, keepdims=True)

    return body


def _make_cls_kernel(F):
    P1, P2 = _plen(F), _plen(_plen(F))

    def body(x_ref, w1_ref, h1_ref, w2_ref, h2_ref, bp2_ref,
             w3_ref, h3_ref, bp3_ref, wf_ref, bf_ref, wf1_ref, bf1_ref,
             o_ref,
             pad1, act1, pad2, act2, pad3, act3):
        cin = x_ref.shape[2]
        pad1[0:2, :] = jnp.zeros((2, cin), jnp.float32)
        pad1[F + 2:F + 4, :] = jnp.zeros((2, cin), jnp.float32)
        pad1[2:F + 2, :] = x_ref[0]
        a1 = jnp.dot(_im2col5(pad1, F), w1_ref[...],
                     preferred_element_type=jnp.float32)
        act1[...] = jnp.maximum(a1 + h1_ref[...], 0.0)

        c1 = act1.shape[1]
        pad2[0:2, :] = jnp.zeros((2, c1), jnp.float32)
        pad2[P1 + 2:P1 + 4, :] = jnp.zeros((2, c1), jnp.float32)
        pad2[2:P1 + 2, :] = _pool5s2(act1, F)
        big = jnp.dot(_im2col5(pad2, P1), w2_ref[...],
                      preferred_element_type=jnp.float32)
        c2 = big.shape[1] // 2
        act2[...] = (big[:, c2:] + bp2_ref[...]
                     + jnp.maximum(big[:, :c2] + h2_ref[...], 0.0))

        pad3[0:2, :] = jnp.zeros((2, c2), jnp.float32)
        pad3[P2 + 2:P2 + 4, :] = jnp.zeros((2, c2), jnp.float32)
        pad3[2:P2 + 2, :] = _pool5s2(act2, P1)
        big3 = jnp.dot(_im2col5(pad3, P2), w3_ref[...],
                       preferred_element_type=jnp.float32)
        c3 = big3.shape[1] // 2
        act3[...] = (big3[:, c3:] + bp3_ref[...]
                     + jnp.maximum(big3[:, :c3] + h3_ref[...], 0.0))

        feat = jnp.mean(_pool5s2(act3, P2), axis=0, keepdims=True)
        h = jnp.dot(feat, wf_ref[...],
                    preferred_element_type=jnp.float32) + bf_ref[...]
        y = jnp.dot(h, wf1_ref[...],
                    preferred_element_type=jnp.float32) + bf1_ref[...]
        o_ref[0] = y

    return body


def _cspec(shape):
    nd = len(shape)
    return pl.BlockSpec(shape, (lambda i: (0,) * nd))


def kernel(x, enc_conv1_w, enc_conv1_s, enc_conv1_h,
           enc_conv2_w, enc_conv2_s, enc_conv2_h,
           enc_conv3_w, enc_conv3_s, enc_conv3_h,
           enc_pc2_w, enc_pc2_b, enc_pc3_w, enc_pc3_b,
           cls_conv1_w, cls_conv1_s, cls_conv1_h,
           cls_conv2_w, cls_conv2_s, cls_conv2_h,
           cls_conv3_w, cls_conv3_s, cls_conv3_h,
           cls_pc2_w, cls_pc2_b, cls_pc3_w, cls_pc3_b,
           cls_fc_w, cls_fc_b, cls_fc1_w, cls_fc1_b):
    B, nlead, L = x.shape
    N = B * nlead
    EC1 = enc_conv1_w.shape[2]
    EC2 = enc_conv2_w.shape[2]
    EC3 = enc_conv3_w.shape[2]
    P1, P2 = _plen(L), _plen(_plen(L))
    P3 = _plen(P2)

    ew1 = (enc_conv1_w.reshape(5, EC1) * enc_conv1_s)
    ew2 = _fuse_conv_point(enc_conv2_w, enc_conv2_s, enc_pc2_w)
    ew3 = _fuse_conv_point(enc_conv3_w, enc_conv3_s, enc_pc3_w)

    leads = x.reshape(N, L, 1)
    feat = pl.pallas_call(
        _make_enc_kernel(L),
        out_shape=jax.ShapeDtypeStruct((N, 1, EC3), jnp.float32),
        grid=(N,),
        in_specs=[
            pl.BlockSpec((1, L, 1), lambda i: (i, 0, 0)),
            _cspec(ew1.shape), _cspec((1, EC1)),
            _cspec(ew2.shape), _cspec((1, EC2)), _cspec((1, EC2)),
            _cspec(ew3.shape), _cspec((1, EC3)), _cspec((1, EC3)),
        ],
        out_specs=pl.BlockSpec((1, 1, EC3), lambda i: (i, 0, 0)),
        scratch_shapes=[
            pltpu.VMEM((L + 4, 1), jnp.float32),
            pltpu.VMEM((L, EC1), jnp.float32),
            pltpu.VMEM((P1 + 4, EC1), jnp.float32),
            pltpu.VMEM((P1, EC2), jnp.float32),
            pltpu.VMEM((P2 + 4, EC2), jnp.float32),
            pltpu.VMEM((P2, EC3), jnp.float32),
        ],
        compiler_params=pltpu.CompilerParams(
            dimension_semantics=("parallel",)),
    )(leads, ew1, enc_conv1_h, ew2, enc_conv2_h, enc_pc2_b,
      ew3, enc_conv3_h, enc_pc3_b)

    F = EC3
    CC1 = cls_conv1_w.shape[2]
    CC2 = cls_conv2_w.shape[2]
    CC3 = cls_conv3_w.shape[2]
    Q1, Q2 = _plen(F), _plen(_plen(F))
    H = cls_fc_w.shape[1]
    NC = cls_fc1_w.shape[1]
    Hp = ((H + 127) // 128) * 128
    wf, bf, wf1 = cls_fc_w, cls_fc_b, cls_fc1_w
    if Hp != H:
        wf = jnp.pad(wf, ((0, 0), (0, Hp - H)))
        bf = jnp.pad(bf, ((0, 0), (0, Hp - H)))
        wf1 = jnp.pad(wf1, ((0, Hp - H), (0, 0)))

    cw1 = ((cls_conv1_w * cls_conv1_s)
           .reshape(5 * nlead, CC1).astype(jnp.bfloat16))
    cw2 = _fuse_conv_point(cls_conv2_w, cls_conv2_s, cls_pc2_w)
    cw3 = _fuse_conv_point(cls_conv3_w, cls_conv3_s, cls_pc3_w)

    cls_in = feat.reshape(B, nlead, EC3).transpose(0, 2, 1)

    out = pl.pallas_call(
        _make_cls_kernel(F),
        out_shape=jax.ShapeDtypeStruct((B, 1, NC), jnp.float32),
        grid=(B,),
        in_specs=[
            pl.BlockSpec((1, F, nlead), lambda i: (i, 0, 0)),
            _cspec(cw1.shape), _cspec((1, CC1)),
            _cspec(cw2.shape), _cspec((1, CC2)), _cspec((1, CC2)),
            _cspec(cw3.shape), _cspec((1, CC3)), _cspec((1, CC3)),
            _cspec((F, Hp)), _cspec((1, Hp)),
            _cspec((Hp, NC)), _cspec((1, NC)),
        ],
        out_specs=pl.BlockSpec((1, 1, NC), lambda i: (i, 0, 0)),
        scratch_shapes=[
            pltpu.VMEM((F + 4, nlead), jnp.float32),
            pltpu.VMEM((F, CC1), jnp.float32),
            pltpu.VMEM((Q1 + 4, CC1), jnp.float32),
            pltpu.VMEM((Q1, CC2), jnp.float32),
            pltpu.VMEM((Q2 + 4, CC2), jnp.float32),
            pltpu.VMEM((Q2, CC3), jnp.float32),
        ],
        compiler_params=pltpu.CompilerParams(
            dimension_semantics=("parallel",)),
    )(cls_in, cw1, cls_conv1_h, cw2, cls_conv2_h, cls_pc2_b,
      cw3, cls_conv3_h, cls_pc3_b, wf, bf, wf1, cls_fc1_b)
    return out[:, 0, :]


# speedup vs baseline: 2.2749x; 2.0870x over previous
import jax
import jax.numpy as jnp
from jax.experimental import pallas as pl
from jax.experimental.pallas import tpu as pltpu


def _plen(l):
    return (l - 5) // 2 + 1


def _pool5s2(ref, lin):
    p = _plen(lin)
    m = ref[pl.ds(0, p, 2), :]
    for k in range(1, 5):
        m = jnp.maximum(m, ref[pl.ds(k, p, 2), :])
    return m


def _im2col5(pad_ref, lout):
    cols = [pad_ref[k:k + lout, :] for k in range(5)]
    return jnp.concatenate(cols, axis=1).astype(jnp.bfloat16)


def _fuse_conv_point(wc, scale, wp):
    cin, co = wc.shape[1], wc.shape[2]
    conv_part = (wc * scale).reshape(5 * cin, co)
    point_part = jnp.zeros((5 * cin, wp.shape[1]), jnp.float32)
    point_part = point_part.at[2 * cin:3 * cin].set(wp)
    return jnp.concatenate([conv_part, point_part], axis=1).astype(jnp.bfloat16)


def _make_enc_kernel(L):
    P1, P2 = _plen(L), _plen(_plen(L))

    LP = L + 12

    def body(x_ref, w1_ref, h1_ref, w2_ref, h2_ref, bp2_ref,
             w3_ref, h3_ref, bp3_ref, o_ref,
             taps, act1, pad2, act2, pad3, act3):
        xv = x_ref[0]
        for k in range(5):
            taps[k:k + 1, :] = jnp.concatenate(
                [xv[:, 6 + k:], jnp.zeros((1, 6 + k), jnp.float32)], axis=1)
        actT = jnp.dot(w1_ref[...], taps[...],
                       preferred_element_type=jnp.float32)
        actT = jnp.maximum(actT + h1_ref[...], 0.0)
        act1[...] = actT[:, :L].T

        c1 = w1_ref.shape[0]
        pad2[0:2, :] = jnp.zeros((2, c1), jnp.float32)
        pad2[P1 + 2:P1 + 4, :] = jnp.zeros((2, c1), jnp.float32)
        pad2[2:P1 + 2, :] = _pool5s2(act1, L)
        big = jnp.dot(_im2col5(pad2, P1), w2_ref[...],
                      preferred_element_type=jnp.float32)
        c2 = big.shape[1] // 2
        act2[...] = (big[:, c2:] + bp2_ref[...]
                     + jnp.maximum(big[:, :c2] + h2_ref[...], 0.0))

        pad3[0:2, :] = jnp.zeros((2, c2), jnp.float32)
        pad3[P2 + 2:P2 + 4, :] = jnp.zeros((2, c2), jnp.float32)
        pad3[2:P2 + 2, :] = _pool5s2(act2, P1)
        big3 = jnp.dot(_im2col5(pad3, P2), w3_ref[...],
                       preferred_element_type=jnp.float32)
        c3 = big3.shape[1] // 2
        act3[...] = (big3[:, c3:] + bp3_ref[...]
                     + jnp.maximum(big3[:, :c3] + h3_ref[...], 0.0))

        o_ref[0] = jnp.mean(_pool5s2(act3, P2), axis=0, keepdims=True)

    return body


def _make_cls_kernel(F):
    P1, P2 = _plen(F), _plen(_plen(F))

    def body(x_ref, w1_ref, h1_ref, w2_ref, h2_ref, bp2_ref,
             w3_ref, h3_ref, bp3_ref, wf_ref, bf_ref, wf1_ref, bf1_ref,
             o_ref,
             pad1, act1, pad2, act2, pad3, act3):
        cin = x_ref.shape[2]
        pad1[0:2, :] = jnp.zeros((2, cin), jnp.float32)
        pad1[F + 2:F + 4, :] = jnp.zeros((2, cin), jnp.float32)
        pad1[2:F + 2, :] = x_ref[0]
        a1 = jnp.dot(_im2col5(pad1, F), w1_ref[...],
                     preferred_element_type=jnp.float32)
        act1[...] = jnp.maximum(a1 + h1_ref[...], 0.0)

        c1 = act1.shape[1]
        pad2[0:2, :] = jnp.zeros((2, c1), jnp.float32)
        pad2[P1 + 2:P1 + 4, :] = jnp.zeros((2, c1), jnp.float32)
        pad2[2:P1 + 2, :] = _pool5s2(act1, F)
        big = jnp.dot(_im2col5(pad2, P1), w2_ref[...],
                      preferred_element_type=jnp.float32)
        c2 = big.shape[1] // 2
        act2[...] = (big[:, c2:] + bp2_ref[...]
                     + jnp.maximum(big[:, :c2] + h2_ref[...], 0.0))

        pad3[0:2, :] = jnp.zeros((2, c2), jnp.float32)
        pad3[P2 + 2:P2 + 4, :] = jnp.zeros((2, c2), jnp.float32)
        pad3[2:P2 + 2, :] = _pool5s2(act2, P1)
        big3 = jnp.dot(_im2col5(pad3, P2), w3_ref[...],
                       preferred_element_type=jnp.float32)
        c3 = big3.shape[1] // 2
        act3[...] = (big3[:, c3:] + bp3_ref[...]
                     + jnp.maximum(big3[:, :c3] + h3_ref[...], 0.0))

        feat = jnp.mean(_pool5s2(act3, P2), axis=0, keepdims=True)
        h = jnp.dot(feat, wf_ref[...],
                    preferred_element_type=jnp.float32) + bf_ref[...]
        y = jnp.dot(h, wf1_ref[...],
                    preferred_element_type=jnp.float32) + bf1_ref[...]
        o_ref[0] = y

    return body


def _cspec(shape):
    nd = len(shape)
    return pl.BlockSpec(shape, (lambda i: (0,) * nd))


def kernel(x, enc_conv1_w, enc_conv1_s, enc_conv1_h,
           enc_conv2_w, enc_conv2_s, enc_conv2_h,
           enc_conv3_w, enc_conv3_s, enc_conv3_h,
           enc_pc2_w, enc_pc2_b, enc_pc3_w, enc_pc3_b,
           cls_conv1_w, cls_conv1_s, cls_conv1_h,
           cls_conv2_w, cls_conv2_s, cls_conv2_h,
           cls_conv3_w, cls_conv3_s, cls_conv3_h,
           cls_pc2_w, cls_pc2_b, cls_pc3_w, cls_pc3_b,
           cls_fc_w, cls_fc_b, cls_fc1_w, cls_fc1_b):
    B, nlead, L = x.shape
    N = B * nlead
    EC1 = enc_conv1_w.shape[2]
    EC2 = enc_conv2_w.shape[2]
    EC3 = enc_conv3_w.shape[2]
    P1, P2 = _plen(L), _plen(_plen(L))
    P3 = _plen(P2)

    ew1 = (enc_conv1_w.reshape(5, EC1) * enc_conv1_s).T
    eh1 = enc_conv1_h.T
    ew2 = _fuse_conv_point(enc_conv2_w, enc_conv2_s, enc_pc2_w)
    ew3 = _fuse_conv_point(enc_conv3_w, enc_conv3_s, enc_pc3_w)

    leads = jnp.pad(x.reshape(N, L), ((0, 0), (8, 4))).reshape(N, 1, L + 12)
    feat = pl.pallas_call(
        _make_enc_kernel(L),
        out_shape=jax.ShapeDtypeStruct((N, 1, EC3), jnp.float32),
        grid=(N,),
        in_specs=[
            pl.BlockSpec((1, 1, L + 12), lambda i: (i, 0, 0)),
            _cspec(ew1.shape), _cspec((EC1, 1)),
            _cspec(ew2.shape), _cspec((1, EC2)), _cspec((1, EC2)),
            _cspec(ew3.shape), _cspec((1, EC3)), _cspec((1, EC3)),
        ],
        out_specs=pl.BlockSpec((1, 1, EC3), lambda i: (i, 0, 0)),
        scratch_shapes=[
            pltpu.VMEM((5, L + 12), jnp.float32),
            pltpu.VMEM((L, EC1), jnp.float32),
            pltpu.VMEM((P1 + 4, EC1), jnp.float32),
            pltpu.VMEM((P1, EC2), jnp.float32),
            pltpu.VMEM((P2 + 4, EC2), jnp.float32),
            pltpu.VMEM((P2, EC3), jnp.float32),
        ],
        compiler_params=pltpu.CompilerParams(
            dimension_semantics=("parallel",)),
    )(leads, ew1, eh1, ew2, enc_conv2_h, enc_pc2_b,
      ew3, enc_conv3_h, enc_pc3_b)

    F = EC3
    CC1 = cls_conv1_w.shape[2]
    CC2 = cls_conv2_w.shape[2]
    CC3 = cls_conv3_w.shape[2]
    Q1, Q2 = _plen(F), _plen(_plen(F))
    H = cls_fc_w.shape[1]
    NC = cls_fc1_w.shape[1]
    Hp = ((H + 127) // 128) * 128
    wf, bf, wf1 = cls_fc_w, cls_fc_b, cls_fc1_w
    if Hp != H:
        wf = jnp.pad(wf, ((0, 0), (0, Hp - H)))
        bf = jnp.pad(bf, ((0, 0), (0, Hp - H)))
        wf1 = jnp.pad(wf1, ((0, Hp - H), (0, 0)))

    cw1 = ((cls_conv1_w * cls_conv1_s)
           .reshape(5 * nlead, CC1).astype(jnp.bfloat16))
    cw2 = _fuse_conv_point(cls_conv2_w, cls_conv2_s, cls_pc2_w)
    cw3 = _fuse_conv_point(cls_conv3_w, cls_conv3_s, cls_pc3_w)

    cls_in = feat.reshape(B, nlead, EC3).transpose(0, 2, 1)

    out = pl.pallas_call(
        _make_cls_kernel(F),
        out_shape=jax.ShapeDtypeStruct((B, 1, NC), jnp.float32),
        grid=(B,),
        in_specs=[
            pl.BlockSpec((1, F, nlead), lambda i: (i, 0, 0)),
            _cspec(cw1.shape), _cspec((1, CC1)),
            _cspec(cw2.shape), _cspec((1, CC2)), _cspec((1, CC2)),
            _cspec(cw3.shape), _cspec((1, CC3)), _cspec((1, CC3)),
            _cspec((F, Hp)), _cspec((1, Hp)),
            _cspec((Hp, NC)), _cspec((1, NC)),
        ],
        out_specs=pl.BlockSpec((1, 1, NC), lambda i: (i, 0, 0)),
        scratch_shapes=[
            pltpu.VMEM((F + 4, nlead), jnp.float32),
            pltpu.VMEM((F, CC1), jnp.float32),
            pltpu.VMEM((Q1 + 4, CC1), jnp.float32),
            pltpu.VMEM((Q1, CC2), jnp.float32),
            pltpu.VMEM((Q2 + 4, CC2), jnp.float32),
            pltpu.VMEM((Q2, CC3), jnp.float32),
        ],
        compiler_params=pltpu.CompilerParams(
            dimension_semantics=("parallel",)),
    )(cls_in, cw1, cls_conv1_h, cw2, cls_conv2_h, cls_pc2_b,
      cw3, cls_conv3_h, cls_pc3_b, wf, bf, wf1, cls_fc1_b)
    return out[:, 0, :]


# speedup vs baseline: 2.4497x; 1.0769x over previous
import jax
import jax.numpy as jnp
from jax.experimental import pallas as pl
from jax.experimental.pallas import tpu as pltpu


def _plen(l):
    return (l - 5) // 2 + 1


def _shl(v, s):
    r, t = v.shape
    return jnp.concatenate([v[:, s:], jnp.zeros((r, s), v.dtype)], axis=1)


def _pool5s2(ref, base, lin):
    p = _plen(lin)
    m = ref[pl.ds(base, p, 2), :]
    for k in range(1, 5):
        m = jnp.maximum(m, ref[pl.ds(base + k, p, 2), :])
    return m


def _im2col5(pad_ref, lout):
    cols = [pad_ref[k:k + lout, :] for k in range(5)]
    return jnp.concatenate(cols, axis=1).astype(jnp.bfloat16)


def _fuse_conv_point(wc, scale, wp):
    cin, co = wc.shape[1], wc.shape[2]
    conv_part = (wc * scale).reshape(5 * cin, co)
    point_part = jnp.zeros((5 * cin, wp.shape[1]), jnp.float32)
    point_part = point_part.at[2 * cin:3 * cin].set(wp)
    return jnp.concatenate([conv_part, point_part], axis=1).astype(jnp.bfloat16)


def _make_enc_kernel(L, S1, G):
    P1, P2 = _plen(L), _plen(_plen(L))

    def body(x_ref, w1_ref, h1_ref, w2_ref, h2_ref, bp2_ref,
             w3_ref, h3_ref, bp3_ref, o_ref,
             taps, act1, pad2, act2, pad3, act3):
        xv = x_ref[0]
        for k in range(5):
            taps[k:k + 1, :] = _shl(xv, 6 + k)
        actT = jnp.dot(w1_ref[...], taps[...],
                       preferred_element_type=jnp.float32)
        actT = jnp.maximum(actT + h1_ref[...], 0.0)
        act1[...] = actT.T

        c1 = w1_ref.shape[0]
        feats = []
        for g in range(G):
            pad2[0:2, :] = jnp.zeros((2, c1), jnp.float32)
            pad2[P1 + 2:P1 + 4, :] = jnp.zeros((2, c1), jnp.float32)
            pad2[2:P1 + 2, :] = _pool5s2(act1, g * S1, L)
            big = jnp.dot(_im2col5(pad2, P1), w2_ref[...],
                          preferred_element_type=jnp.float32)
            c2 = big.shape[1] // 2
            act2[...] = (big[:, c2:] + bp2_ref[...]
                         + jnp.maximum(big[:, :c2] + h2_ref[...], 0.0))

            pad3[0:2, :] = jnp.zeros((2, c2), jnp.float32)
            pad3[P2 + 2:P2 + 4, :] = jnp.zeros((2, c2), jnp.float32)
            pad3[2:P2 + 2, :] = _pool5s2(act2, 0, P1)
            big3 = jnp.dot(_im2col5(pad3, P2), w3_ref[...],
                           preferred_element_type=jnp.float32)
            c3 = big3.shape[1] // 2
            act3[...] = (big3[:, c3:] + bp3_ref[...]
                         + jnp.maximum(big3[:, :c3] + h3_ref[...], 0.0))

            feats.append(jnp.mean(_pool5s2(act3, 0, P2),
                                  axis=0, keepdims=True))
        o_ref[0] = jnp.concatenate(feats, axis=0)

    return body


def _make_cls_kernel(F):
    P1, P2 = _plen(F), _plen(_plen(F))

    def body(x_ref, w1_ref, h1_ref, w2_ref, h2_ref, bp2_ref,
             w3_ref, h3_ref, bp3_ref, wf_ref, bf_ref, wf1_ref, bf1_ref,
             o_ref,
             pad1, act1, pad2, act2, pad3, act3):
        cin = x_ref.shape[2]
        pad1[0:2, :] = jnp.zeros((2, cin), jnp.float32)
        pad1[F + 2:F + 4, :] = jnp.zeros((2, cin), jnp.float32)
        pad1[2:F + 2, :] = x_ref[0]
        a1 = jnp.dot(_im2col5(pad1, F), w1_ref[...],
                     preferred_element_type=jnp.float32)
        act1[...] = jnp.maximum(a1 + h1_ref[...], 0.0)

        c1 = act1.shape[1]
        pad2[0:2, :] = jnp.zeros((2, c1), jnp.float32)
        pad2[P1 + 2:P1 + 4, :] = jnp.zeros((2, c1), jnp.float32)
        pad2[2:P1 + 2, :] = _pool5s2(act1, 0, F)
        big = jnp.dot(_im2col5(pad2, P1), w2_ref[...],
                      preferred_element_type=jnp.float32)
        c2 = big.shape[1] // 2
        act2[...] = (big[:, c2:] + bp2_ref[...]
                     + jnp.maximum(big[:, :c2] + h2_ref[...], 0.0))

        pad3[0:2, :] = jnp.zeros((2, c2), jnp.float32)
        pad3[P2 + 2:P2 + 4, :] = jnp.zeros((2, c2), jnp.float32)
        pad3[2:P2 + 2, :] = _pool5s2(act2, 0, P1)
        big3 = jnp.dot(_im2col5(pad3, P2), w3_ref[...],
                       preferred_element_type=jnp.float32)
        c3 = big3.shape[1] // 2
        act3[...] = (big3[:, c3:] + bp3_ref[...]
                     + jnp.maximum(big3[:, :c3] + h3_ref[...], 0.0))

        feat = jnp.mean(_pool5s2(act3, 0, P2), axis=0, keepdims=True)
        h = jnp.dot(feat, wf_ref[...],
                    preferred_element_type=jnp.float32) + bf_ref[...]
        y = jnp.dot(h, wf1_ref[...],
                    preferred_element_type=jnp.float32) + bf1_ref[...]
        o_ref[0] = y

    return body


def _cspec(shape):
    nd = len(shape)
    return pl.BlockSpec(shape, (lambda i: (0,) * nd))


def kernel(x, enc_conv1_w, enc_conv1_s, enc_conv1_h,
           enc_conv2_w, enc_conv2_s, enc_conv2_h,
           enc_conv3_w, enc_conv3_s, enc_conv3_h,
           enc_pc2_w, enc_pc2_b, enc_pc3_w, enc_pc3_b,
           cls_conv1_w, cls_conv1_s, cls_conv1_h,
           cls_conv2_w, cls_conv2_s, cls_conv2_h,
           cls_conv3_w, cls_conv3_s, cls_conv3_h,
           cls_pc2_w, cls_pc2_b, cls_pc3_w, cls_pc3_b,
           cls_fc_w, cls_fc_b, cls_fc1_w, cls_fc1_b):
    B, nlead, L = x.shape
    N = B * nlead
    EC1 = enc_conv1_w.shape[2]
    EC2 = enc_conv2_w.shape[2]
    EC3 = enc_conv3_w.shape[2]
    P1, P2 = _plen(L), _plen(_plen(L))

    G = 4
    S1 = ((L + 16 + 7) // 8) * 8

    ew1 = (enc_conv1_w.reshape(5, EC1) * enc_conv1_s).T
    eh1 = enc_conv1_h.T
    ew2 = _fuse_conv_point(enc_conv2_w, enc_conv2_s, enc_pc2_w)
    ew3 = _fuse_conv_point(enc_conv3_w, enc_conv3_s, enc_pc3_w)

    leads = jnp.pad(x.reshape(N, L), ((0, 0), (8, S1 - L - 8)))
    leads = leads.reshape(N // G, 1, G * S1)

    feat = pl.pallas_call(
        _make_enc_kernel(L, S1, G),
        out_shape=jax.ShapeDtypeStruct((N // G, G, EC3), jnp.float32),
        grid=(N // G,),
        in_specs=[
            pl.BlockSpec((1, 1, G * S1), lambda i: (i, 0, 0)),
            _cspec((EC1, 5)), _cspec((EC1, 1)),
            _cspec(ew2.shape), _cspec((1, EC2)), _cspec((1, EC2)),
            _cspec(ew3.shape), _cspec((1, EC3)), _cspec((1, EC3)),
        ],
        out_specs=pl.BlockSpec((1, G, EC3), lambda i: (i, 0, 0)),
        scratch_shapes=[
            pltpu.VMEM((5, G * S1), jnp.float32),
            pltpu.VMEM((G * S1, EC1), jnp.float32),
            pltpu.VMEM((P1 + 4, EC1), jnp.float32),
            pltpu.VMEM((P1, EC2), jnp.float32),
            pltpu.VMEM((P2 + 4, EC2), jnp.float32),
            pltpu.VMEM((P2, EC3), jnp.float32),
        ],
        compiler_params=pltpu.CompilerParams(
            dimension_semantics=("parallel",)),
    )(leads, ew1, eh1, ew2, enc_conv2_h, enc_pc2_b,
      ew3, enc_conv3_h, enc_pc3_b)

    feat = feat.reshape(N, EC3)

    F = EC3
    CC1 = cls_conv1_w.shape[2]
    CC2 = cls_conv2_w.shape[2]
    CC3 = cls_conv3_w.shape[2]
    Q1, Q2 = _plen(F), _plen(_plen(F))
    H = cls_fc_w.shape[1]
    NC = cls_fc1_w.shape[1]
    Hp = ((H + 127) // 128) * 128
    wf, bf, wf1 = cls_fc_w, cls_fc_b, cls_fc1_w
    if Hp != H:
        wf = jnp.pad(wf, ((0, 0), (0, Hp - H)))
        bf = jnp.pad(bf, ((0, 0), (0, Hp - H)))
        wf1 = jnp.pad(wf1, ((0, Hp - H), (0, 0)))

    cw1 = ((cls_conv1_w * cls_conv1_s)
           .reshape(5 * nlead, CC1).astype(jnp.bfloat16))
    cw2 = _fuse_conv_point(cls_conv2_w, cls_conv2_s, cls_pc2_w)
    cw3 = _fuse_conv_point(cls_conv3_w, cls_conv3_s, cls_pc3_w)

    cls_in = feat.reshape(B, nlead, EC3).transpose(0, 2, 1)

    out = pl.pallas_call(
        _make_cls_kernel(F),
        out_shape=jax.ShapeDtypeStruct((B, 1, NC), jnp.float32),
        grid=(B,),
        in_specs=[
            pl.BlockSpec((1, F, nlead), lambda i: (i, 0, 0)),
            _cspec(cw1.shape), _cspec((1, CC1)),
            _cspec(cw2.shape), _cspec((1, CC2)), _cspec((1, CC2)),
            _cspec(cw3.shape), _cspec((1, CC3)), _cspec((1, CC3)),
            _cspec((F, Hp)), _cspec((1, Hp)),
            _cspec((Hp, NC)), _cspec((1, NC)),
        ],
        out_specs=pl.BlockSpec((1, 1, NC), lambda i: (i, 0, 0)),
        scratch_shapes=[
            pltpu.VMEM((F + 4, nlead), jnp.float32),
            pltpu.VMEM((F, CC1), jnp.float32),
            pltpu.VMEM((Q1 + 4, CC1), jnp.float32),
            pltpu.VMEM((Q1, CC2), jnp.float32),
            pltpu.VMEM((Q2 + 4, CC2), jnp.float32),
            pltpu.VMEM((Q2, CC3), jnp.float32),
        ],
        compiler_params=pltpu.CompilerParams(
            dimension_semantics=("parallel",)),
    )(cls_in, cw1, cls_conv1_h, cw2, cls_conv2_h, cls_pc2_b,
      cw3, cls_conv3_h, cls_pc3_b, wf, bf, wf1, cls_fc1_b)
    return out[:, 0, :]


# speedup vs baseline: 2.4751x; 1.0104x over previous
import jax
import jax.numpy as jnp
from jax.experimental import pallas as pl
from jax.experimental.pallas import tpu as pltpu


def _plen(l):
    return (l - 5) // 2 + 1


def _shl(v, s):
    r, t = v.shape
    return jnp.concatenate([v[:, s:], jnp.zeros((r, s), v.dtype)], axis=1)


def _pool5s2(ref, base, lin):
    p = _plen(lin)
    m = ref[pl.ds(base, p, 2), :]
    for k in range(1, 5):
        m = jnp.maximum(m, ref[pl.ds(base + k, p, 2), :])
    return m


def _im2col5(pad_ref, lout):
    cols = [pad_ref[k:k + lout, :] for k in range(5)]
    return jnp.concatenate(cols, axis=1).astype(jnp.bfloat16)


def _fuse_conv_point(wc, scale, wp):
    cin, co = wc.shape[1], wc.shape[2]
    conv_part = (wc * scale).reshape(5 * cin, co)
    point_part = jnp.zeros((5 * cin, wp.shape[1]), jnp.float32)
    point_part = point_part.at[2 * cin:3 * cin].set(wp)
    return jnp.concatenate([conv_part, point_part], axis=1).astype(jnp.bfloat16)


def _make_enc_kernel(L, S1, G):
    P1, P2 = _plen(L), _plen(_plen(L))

    def body(x_ref, w1_ref, h1_ref, w2_ref, h2_ref, bp2_ref,
             w3_ref, h3_ref, bp3_ref, o_ref,
             taps, act1, pad2, act2, pad3, act3):
        xv = x_ref[0]
        for k in range(5):
            taps[k:k + 1, :] = _shl(xv, 6 + k)
        actT = jnp.dot(w1_ref[...], taps[...],
                       preferred_element_type=jnp.float32)
        actT = jnp.maximum(actT + h1_ref[...], 0.0)
        act1[...] = actT.T

        c1 = w1_ref.shape[0]
        feats = []
        for g in range(G):
            pad2[0:2, :] = jnp.zeros((2, c1), jnp.float32)
            pad2[P1 + 2:P1 + 4, :] = jnp.zeros((2, c1), jnp.float32)
            pad2[2:P1 + 2, :] = _pool5s2(act1, g * S1, L)
            big = jnp.dot(_im2col5(pad2, P1), w2_ref[...],
                          preferred_element_type=jnp.float32)
            c2 = big.shape[1] // 2
            act2[...] = (big[:, c2:] + bp2_ref[...]
                         + jnp.maximum(big[:, :c2] + h2_ref[...], 0.0))

            pad3[0:2, :] = jnp.zeros((2, c2), jnp.float32)
            pad3[P2 + 2:P2 + 4, :] = jnp.zeros((2, c2), jnp.float32)
            pad3[2:P2 + 2, :] = _pool5s2(act2, 0, P1)
            big3 = jnp.dot(_im2col5(pad3, P2), w3_ref[...],
                           preferred_element_type=jnp.float32)
            c3 = big3.shape[1] // 2
            act3[...] = (big3[:, c3:] + bp3_ref[...]
                         + jnp.maximum(big3[:, :c3] + h3_ref[...], 0.0))

            feats.append(jnp.mean(_pool5s2(act3, 0, P2),
                                  axis=0, keepdims=True))
        o_ref[0] = jnp.concatenate(feats, axis=0)

    return body


def _make_cls_kernel(F, Gc):
    P1, P2 = _plen(F), _plen(_plen(F))

    def body(x_ref, w1_ref, h1_ref, w2_ref, h2_ref, bp2_ref,
             w3_ref, h3_ref, bp3_ref, wf_ref, bf_ref, wf1_ref, bf1_ref,
             o_ref,
             pad1, act1, pad2, act2, pad3, act3):
        cin = x_ref.shape[2]
        feats = []
        for g in range(Gc):
            pad1[0:2, :] = jnp.zeros((2, cin), jnp.float32)
            pad1[F + 2:F + 4, :] = jnp.zeros((2, cin), jnp.float32)
            pad1[2:F + 2, :] = x_ref[g]
            a1 = jnp.dot(_im2col5(pad1, F), w1_ref[...],
                         preferred_element_type=jnp.float32)
            act1[...] = jnp.maximum(a1 + h1_ref[...], 0.0)

            c1 = act1.shape[1]
            pad2[0:2, :] = jnp.zeros((2, c1), jnp.float32)
            pad2[P1 + 2:P1 + 4, :] = jnp.zeros((2, c1), jnp.float32)
            pad2[2:P1 + 2, :] = _pool5s2(act1, 0, F)
            big = jnp.dot(_im2col5(pad2, P1), w2_ref[...],
                          preferred_element_type=jnp.float32)
            c2 = big.shape[1] // 2
            act2[...] = (big[:, c2:] + bp2_ref[...]
                         + jnp.maximum(big[:, :c2] + h2_ref[...], 0.0))

            pad3[0:2, :] = jnp.zeros((2, c2), jnp.float32)
            pad3[P2 + 2:P2 + 4, :] = jnp.zeros((2, c2), jnp.float32)
            pad3[2:P2 + 2, :] = _pool5s2(act2, 0, P1)
            big3 = jnp.dot(_im2col5(pad3, P2), w3_ref[...],
                           preferred_element_type=jnp.float32)
            c3 = big3.shape[1] // 2
            act3[...] = (big3[:, c3:] + bp3_ref[...]
                         + jnp.maximum(big3[:, :c3] + h3_ref[...], 0.0))

            feats.append(jnp.mean(_pool5s2(act3, 0, P2),
                                  axis=0, keepdims=True))

        feat = jnp.concatenate(feats, axis=0)
        h = jnp.dot(feat, wf_ref[...],
                    preferred_element_type=jnp.float32) + bf_ref[...]
        y = jnp.dot(h, wf1_ref[...],
                    preferred_element_type=jnp.float32) + bf1_ref[...]
        o_ref[0] = y

    return body


def _cspec(shape):
    nd = len(shape)
    return pl.BlockSpec(shape, (lambda i: (0,) * nd))


def kernel(x, enc_conv1_w, enc_conv1_s, enc_conv1_h,
           enc_conv2_w, enc_conv2_s, enc_conv2_h,
           enc_conv3_w, enc_conv3_s, enc_conv3_h,
           enc_pc2_w, enc_pc2_b, enc_pc3_w, enc_pc3_b,
           cls_conv1_w, cls_conv1_s, cls_conv1_h,
           cls_conv2_w, cls_conv2_s, cls_conv2_h,
           cls_conv3_w, cls_conv3_s, cls_conv3_h,
           cls_pc2_w, cls_pc2_b, cls_pc3_w, cls_pc3_b,
           cls_fc_w, cls_fc_b, cls_fc1_w, cls_fc1_b):
    B, nlead, L = x.shape
    N = B * nlead
    EC1 = enc_conv1_w.shape[2]
    EC2 = enc_conv2_w.shape[2]
    EC3 = enc_conv3_w.shape[2]
    P1, P2 = _plen(L), _plen(_plen(L))

    G = 6
    S1 = ((L + 16 + 7) // 8) * 8

    ew1 = (enc_conv1_w.reshape(5, EC1) * enc_conv1_s).T
    eh1 = enc_conv1_h.T
    ew2 = _fuse_conv_point(enc_conv2_w, enc_conv2_s, enc_pc2_w)
    ew3 = _fuse_conv_point(enc_conv3_w, enc_conv3_s, enc_pc3_w)

    leads = jnp.pad(x.reshape(N, L), ((0, 0), (8, S1 - L - 8)))
    leads = leads.reshape(N // G, 1, G * S1)

    feat = pl.pallas_call(
        _make_enc_kernel(L, S1, G),
        out_shape=jax.ShapeDtypeStruct((N // G, G, EC3), jnp.float32),
        grid=(N // G,),
        in_specs=[
            pl.BlockSpec((1, 1, G * S1), lambda i: (i, 0, 0)),
            _cspec((EC1, 5)), _cspec((EC1, 1)),
            _cspec(ew2.shape), _cspec((1, EC2)), _cspec((1, EC2)),
            _cspec(ew3.shape), _cspec((1, EC3)), _cspec((1, EC3)),
        ],
        out_specs=pl.BlockSpec((1, G, EC3), lambda i: (i, 0, 0)),
        scratch_shapes=[
            pltpu.VMEM((5, G * S1), jnp.float32),
            pltpu.VMEM((G * S1, EC1), jnp.float32),
            pltpu.VMEM((P1 + 4, EC1), jnp.float32),
            pltpu.VMEM((P1, EC2), jnp.float32),
            pltpu.VMEM((P2 + 4, EC2), jnp.float32),
            pltpu.VMEM((P2, EC3), jnp.float32),
        ],
        compiler_params=pltpu.CompilerParams(
            dimension_semantics=("parallel",)),
    )(leads, ew1, eh1, ew2, enc_conv2_h, enc_pc2_b,
      ew3, enc_conv3_h, enc_pc3_b)

    feat = feat.reshape(N, EC3)

    F = EC3
    CC1 = cls_conv1_w.shape[2]
    CC2 = cls_conv2_w.shape[2]
    CC3 = cls_conv3_w.shape[2]
    Q1, Q2 = _plen(F), _plen(_plen(F))
    H = cls_fc_w.shape[1]
    NC = cls_fc1_w.shape[1]
    Hp = ((H + 127) // 128) * 128
    wf, bf, wf1 = cls_fc_w, cls_fc_b, cls_fc1_w
    if Hp != H:
        wf = jnp.pad(wf, ((0, 0), (0, Hp - H)))
        bf = jnp.pad(bf, ((0, 0), (0, Hp - H)))
        wf1 = jnp.pad(wf1, ((0, Hp - H), (0, 0)))

    cw1 = ((cls_conv1_w * cls_conv1_s)
           .reshape(5 * nlead, CC1).astype(jnp.bfloat16))
    cw2 = _fuse_conv_point(cls_conv2_w, cls_conv2_s, cls_pc2_w)
    cw3 = _fuse_conv_point(cls_conv3_w, cls_conv3_s, cls_pc3_w)

    cls_in = feat.reshape(B, nlead, EC3).transpose(0, 2, 1)

    Gc = 8
    while B % Gc:
        Gc //= 2
    out = pl.pallas_call(
        _make_cls_kernel(F, Gc),
        out_shape=jax.ShapeDtypeStruct((B // Gc, Gc, NC), jnp.float32),
        grid=(B // Gc,),
        in_specs=[
            pl.BlockSpec((Gc, F, nlead), lambda i: (i, 0, 0)),
            _cspec(cw1.shape), _cspec((1, CC1)),
            _cspec(cw2.shape), _cspec((1, CC2)), _cspec((1, CC2)),
            _cspec(cw3.shape), _cspec((1, CC3)), _cspec((1, CC3)),
            _cspec((F, Hp)), _cspec((1, Hp)),
            _cspec((Hp, NC)), _cspec((1, NC)),
        ],
        out_specs=pl.BlockSpec((1, Gc, NC), lambda i: (i, 0, 0)),
        scratch_shapes=[
            pltpu.VMEM((F + 4, nlead), jnp.float32),
            pltpu.VMEM((F, CC1), jnp.float32),
            pltpu.VMEM((Q1 + 4, CC1), jnp.float32),
            pltpu.VMEM((Q1, CC2), jnp.float32),
            pltpu.VMEM((Q2 + 4, CC2), jnp.float32),
            pltpu.VMEM((Q2, CC3), jnp.float32),
        ],
        compiler_params=pltpu.CompilerParams(
            dimension_semantics=("parallel",)),
    )(cls_in, cw1, cls_conv1_h, cw2, cls_conv2_h, cls_pc2_b,
      cw3, cls_conv3_h, cls_pc3_b, wf, bf, wf1, cls_fc1_b)
    return out.reshape(B, NC)


# speedup vs baseline: 2.7675x; 1.1181x over previous
import jax
import jax.numpy as jnp
from jax.experimental import pallas as pl
from jax.experimental.pallas import tpu as pltpu


def _plen(l):
    return (l - 5) // 2 + 1


def _shl(v, s):
    r, t = v.shape
    return jnp.concatenate([v[:, s:], jnp.zeros((r, s), v.dtype)], axis=1)


def _pool5s2(ref, base, lin):
    p = _plen(lin)
    m = ref[pl.ds(base, p, 2), :]
    for k in range(1, 5):
        m = jnp.maximum(m, ref[pl.ds(base + k, p, 2), :])
    return m


def _im2col5(pad_ref, lout):
    cols = [pad_ref[k:k + lout, :] for k in range(5)]
    return jnp.concatenate(cols, axis=1).astype(jnp.bfloat16)


def _im2col5v(m):
    lout, c = m.shape
    z1 = jnp.zeros((1, c), m.dtype)
    z2 = jnp.zeros((2, c), m.dtype)
    cols = [
        jnp.concatenate([z2, m[:lout - 2]], axis=0),
        jnp.concatenate([z1, m[:lout - 1]], axis=0),
        m,
        jnp.concatenate([m[1:], z1], axis=0),
        jnp.concatenate([m[2:], z2], axis=0),
    ]
    return jnp.concatenate(cols, axis=1).astype(jnp.bfloat16)


def _fuse_conv_point(wc, scale, wp):
    cin, co = wc.shape[1], wc.shape[2]
    conv_part = (wc * scale).reshape(5 * cin, co)
    point_part = jnp.zeros((5 * cin, wp.shape[1]), jnp.float32)
    point_part = point_part.at[2 * cin:3 * cin].set(wp)
    return jnp.concatenate([conv_part, point_part], axis=1).astype(jnp.bfloat16)


def _make_enc_kernel(L, S1, G):
    P1, P2 = _plen(L), _plen(_plen(L))

    def body(x_ref, w1_ref, h1_ref, w2_ref, h2_ref, bp2_ref,
             w3_ref, h3_ref, bp3_ref, o_ref,
             taps, act1, pad2, act2, pad3, act3):
        xv = x_ref[0]
        for k in range(5):
            taps[k:k + 1, :] = _shl(xv, 6 + k)
        actT = jnp.dot(w1_ref[...], taps[...],
                       preferred_element_type=jnp.float32)
        actT = jnp.maximum(actT + h1_ref[...], 0.0)
        act1[...] = actT.T

        c1 = w1_ref.shape[0]
        feats = []
        for g in range(G):
            pad2[0:2, :] = jnp.zeros((2, c1), jnp.bfloat16)
            pad2[P1 + 2:P1 + 4, :] = jnp.zeros((2, c1), jnp.bfloat16)
            pad2[2:P1 + 2, :] = _pool5s2(act1, g * S1, L).astype(jnp.bfloat16)
            big = jnp.dot(_im2col5(pad2, P1), w2_ref[...],
                          preferred_element_type=jnp.float32)
            c2 = big.shape[1] // 2
            act2[...] = (big[:, c2:] + bp2_ref[...]
                         + jnp.maximum(big[:, :c2] + h2_ref[...], 0.0))

            pad3[0:2, :] = jnp.zeros((2, c2), jnp.bfloat16)
            pad3[P2 + 2:P2 + 4, :] = jnp.zeros((2, c2), jnp.bfloat16)
            pad3[2:P2 + 2, :] = _pool5s2(act2, 0, P1).astype(jnp.bfloat16)
            big3 = jnp.dot(_im2col5(pad3, P2), w3_ref[...],
                           preferred_element_type=jnp.float32)
            c3 = big3.shape[1] // 2
            act3[...] = (big3[:, c3:] + bp3_ref[...]
                         + jnp.maximum(big3[:, :c3] + h3_ref[...], 0.0))

            feats.append(jnp.mean(_pool5s2(act3, 0, P2),
                                  axis=0, keepdims=True))
        o_ref[0] = jnp.concatenate(feats, axis=0)

    return body


def _make_cls_kernel(F, Gc):
    P1, P2 = _plen(F), _plen(_plen(F))

    def body(x_ref, w1_ref, h1_ref, w2_ref, h2_ref, bp2_ref,
             w3_ref, h3_ref, bp3_ref, wf_ref, bf_ref, wf1_ref, bf1_ref,
             o_ref,
             pad1, act1, pad2, act2, pad3, act3):
        cin = x_ref.shape[2]
        feats = []
        for g in range(Gc):
            pad1[0:2, :] = jnp.zeros((2, cin), jnp.float32)
            pad1[F + 2:F + 4, :] = jnp.zeros((2, cin), jnp.float32)
            pad1[2:F + 2, :] = x_ref[g]
            a1 = jnp.dot(_im2col5(pad1, F), w1_ref[...],
                         preferred_element_type=jnp.float32)
            act1[...] = jnp.maximum(a1 + h1_ref[...], 0.0)

            c1 = act1.shape[1]
            pad2[0:2, :] = jnp.zeros((2, c1), jnp.float32)
            pad2[P1 + 2:P1 + 4, :] = jnp.zeros((2, c1), jnp.float32)
            pad2[2:P1 + 2, :] = _pool5s2(act1, 0, F)
            big = jnp.dot(_im2col5(pad2, P1), w2_ref[...],
                          preferred_element_type=jnp.float32)
            c2 = big.shape[1] // 2
            act2[...] = (big[:, c2:] + bp2_ref[...]
                         + jnp.maximum(big[:, :c2] + h2_ref[...], 0.0))

            pad3[0:2, :] = jnp.zeros((2, c2), jnp.float32)
            pad3[P2 + 2:P2 + 4, :] = jnp.zeros((2, c2), jnp.float32)
            pad3[2:P2 + 2, :] = _pool5s2(act2, 0, P1)
            big3 = jnp.dot(_im2col5(pad3, P2), w3_ref[...],
                           preferred_element_type=jnp.float32)
            c3 = big3.shape[1] // 2
            act3[...] = (big3[:, c3:] + bp3_ref[...]
                         + jnp.maximum(big3[:, :c3] + h3_ref[...], 0.0))

            feats.append(jnp.mean(_pool5s2(act3, 0, P2),
                                  axis=0, keepdims=True))

        feat = jnp.concatenate(feats, axis=0)
        h = jnp.dot(feat, wf_ref[...],
                    preferred_element_type=jnp.float32) + bf_ref[...]
        y = jnp.dot(h, wf1_ref[...],
                    preferred_element_type=jnp.float32) + bf1_ref[...]
        o_ref[0] = y

    return body


def _cspec(shape):
    nd = len(shape)
    return pl.BlockSpec(shape, (lambda i: (0,) * nd))


def kernel(x, enc_conv1_w, enc_conv1_s, enc_conv1_h,
           enc_conv2_w, enc_conv2_s, enc_conv2_h,
           enc_conv3_w, enc_conv3_s, enc_conv3_h,
           enc_pc2_w, enc_pc2_b, enc_pc3_w, enc_pc3_b,
           cls_conv1_w, cls_conv1_s, cls_conv1_h,
           cls_conv2_w, cls_conv2_s, cls_conv2_h,
           cls_conv3_w, cls_conv3_s, cls_conv3_h,
           cls_pc2_w, cls_pc2_b, cls_pc3_w, cls_pc3_b,
           cls_fc_w, cls_fc_b, cls_fc1_w, cls_fc1_b):
    B, nlead, L = x.shape
    N = B * nlead
    EC1 = enc_conv1_w.shape[2]
    EC2 = enc_conv2_w.shape[2]
    EC3 = enc_conv3_w.shape[2]
    P1, P2 = _plen(L), _plen(_plen(L))

    G = 6
    S1 = ((L + 16 + 7) // 8) * 8

    ew1 = (enc_conv1_w.reshape(5, EC1) * enc_conv1_s).T
    eh1 = enc_conv1_h.T
    ew2 = _fuse_conv_point(enc_conv2_w, enc_conv2_s, enc_pc2_w)
    ew3 = _fuse_conv_point(enc_conv3_w, enc_conv3_s, enc_pc3_w)

    leads = jnp.pad(x.reshape(N, L), ((0, 0), (8, S1 - L - 8)))
    leads = leads.reshape(N // G, 1, G * S1)

    feat = pl.pallas_call(
        _make_enc_kernel(L, S1, G),
        out_shape=jax.ShapeDtypeStruct((N // G, G, EC3), jnp.float32),
        grid=(N // G,),
        in_specs=[
            pl.BlockSpec((1, 1, G * S1), lambda i: (i, 0, 0)),
            _cspec((EC1, 5)), _cspec((EC1, 1)),
            _cspec(ew2.shape), _cspec((1, EC2)), _cspec((1, EC2)),
            _cspec(ew3.shape), _cspec((1, EC3)), _cspec((1, EC3)),
        ],
        out_specs=pl.BlockSpec((1, G, EC3), lambda i: (i, 0, 0)),
        scratch_shapes=[
            pltpu.VMEM((5, G * S1), jnp.float32),
            pltpu.VMEM((G * S1, EC1), jnp.float32),
            pltpu.VMEM((P1 + 4, EC1), jnp.bfloat16),
            pltpu.VMEM((P1, EC2), jnp.float32),
            pltpu.VMEM((P2 + 4, EC2), jnp.bfloat16),
            pltpu.VMEM((P2, EC3), jnp.float32),
        ],
        compiler_params=pltpu.CompilerParams(
            dimension_semantics=("parallel",)),
    )(leads, ew1, eh1, ew2, enc_conv2_h, enc_pc2_b,
      ew3, enc_conv3_h, enc_pc3_b)

    feat = feat.reshape(N, EC3)

    F = EC3
    CC1 = cls_conv1_w.shape[2]
    CC2 = cls_conv2_w.shape[2]
    CC3 = cls_conv3_w.shape[2]
    Q1, Q2 = _plen(F), _plen(_plen(F))
    H = cls_fc_w.shape[1]
    NC = cls_fc1_w.shape[1]
    Hp = ((H + 127) // 128) * 128
    wf, bf, wf1 = cls_fc_w, cls_fc_b, cls_fc1_w
    if Hp != H:
        wf = jnp.pad(wf, ((0, 0), (0, Hp - H)))
        bf = jnp.pad(bf, ((0, 0), (0, Hp - H)))
        wf1 = jnp.pad(wf1, ((0, Hp - H), (0, 0)))

    cw1 = ((cls_conv1_w * cls_conv1_s)
           .reshape(5 * nlead, CC1).astype(jnp.bfloat16))
    cw2 = _fuse_conv_point(cls_conv2_w, cls_conv2_s, cls_pc2_w)
    cw3 = _fuse_conv_point(cls_conv3_w, cls_conv3_s, cls_pc3_w)

    cls_in = feat.reshape(B, nlead, EC3).transpose(0, 2, 1)

    Gc = 8
    while B % Gc:
        Gc //= 2
    out = pl.pallas_call(
        _make_cls_kernel(F, Gc),
        out_shape=jax.ShapeDtypeStruct((B // Gc, Gc, NC), jnp.float32),
        grid=(B // Gc,),
        in_specs=[
            pl.BlockSpec((Gc, F, nlead), lambda i: (i, 0, 0)),
            _cspec(cw1.shape), _cspec((1, CC1)),
            _cspec(cw2.shape), _cspec((1, CC2)), _cspec((1, CC2)),
            _cspec(cw3.shape), _cspec((1, CC3)), _cspec((1, CC3)),
            _cspec((F, Hp)), _cspec((1, Hp)),
            _cspec((Hp, NC)), _cspec((1, NC)),
        ],
        out_specs=pl.BlockSpec((1, Gc, NC), lambda i: (i, 0, 0)),
        scratch_shapes=[
            pltpu.VMEM((F + 4, nlead), jnp.float32),
            pltpu.VMEM((F, CC1), jnp.float32),
            pltpu.VMEM((Q1 + 4, CC1), jnp.float32),
            pltpu.VMEM((Q1, CC2), jnp.float32),
            pltpu.VMEM((Q2 + 4, CC2), jnp.float32),
            pltpu.VMEM((Q2, CC3), jnp.float32),
        ],
        compiler_params=pltpu.CompilerParams(
            dimension_semantics=("parallel",)),
    )(cls_in, cw1, cls_conv1_h, cw2, cls_conv2_h, cls_pc2_b,
      cw3, cls_conv3_h, cls_pc3_b, wf, bf, wf1, cls_fc1_b)
    return out.reshape(B, NC)
